# baseline (device time: 116771 ns/iter reference)
import jax
import jax.numpy as jnp
from jax import lax
from jax.experimental import pallas as pl
from jax.experimental.pallas import tpu as pltpu

N_DEV = 4
M_PER = 1024
HALF = 512
K = 4096
N_PER = 2048
N_HOP = N_DEV - 1
WCH = 512
N_WCH = K // WCH


def kernel(x, w_mat, scale_x, scale_w):
    scale = (scale_x * scale_w).astype(jnp.float32)

    def body(x_hbm, w_hbm, s_ref, out_ref,
             stage_x, x8_ref, stage_w, w8_ref, comm_r, comm_l, stage_out,
             in_sems, w_sems, send_r, recv_r, send_l, recv_l, out_sems):
        my = lax.axis_index("i")
        right = lax.rem(my + 1, N_DEV)
        left = lax.rem(my + N_DEV - 1, N_DEV)

        barrier_sem = pltpu.get_barrier_semaphore()
        pl.semaphore_signal(barrier_sem, inc=1, device_id=(left,),
                            device_id_type=pl.DeviceIdType.MESH)
        pl.semaphore_signal(barrier_sem, inc=1, device_id=(right,),
                            device_id_type=pl.DeviceIdType.MESH)

        cpx = [pltpu.make_async_copy(
                   x_hbm.at[pl.ds(i * HALF, HALF), :],
                   stage_x.at[pl.ds(i * HALF, HALF), :],
                   in_sems.at[i]) for i in range(2)]
        cpx[0].start()
        cpx[1].start()

        cpw = [None] * N_WCH
        for k in range(2):
            cpw[k] = pltpu.make_async_copy(
                w_hbm.at[pl.ds(k * WCH, WCH), :], stage_w.at[k % 2],
                w_sems.at[k % 2])
            cpw[k].start()

        pl.semaphore_wait(barrier_sem, 2)

        rd_r = [None] * N_HOP
        rd_l = [None] * N_HOP
        cpx[0].wait()
        x8_ref[pl.ds(0, HALF), :] = (
            stage_x[pl.ds(0, HALF), :].astype(jnp.float8_e4m3fn))
        rd_r[0] = pltpu.make_async_remote_copy(
            src_ref=x8_ref.at[pl.ds(0, HALF)],
            dst_ref=comm_r.at[0],
            send_sem=send_r.at[0], recv_sem=recv_r.at[0],
            device_id=(right,), device_id_type=pl.DeviceIdType.MESH)
        rd_r[0].start()
        cpx[1].wait()
        x8_ref[pl.ds(HALF, HALF), :] = (
            stage_x[pl.ds(HALF, HALF), :].astype(jnp.float8_e4m3fn))
        rd_l[0] = pltpu.make_async_remote_copy(
            src_ref=x8_ref.at[pl.ds(HALF, HALF)],
            dst_ref=comm_l.at[0],
            send_sem=send_l.at[0], recv_sem=recv_l.at[0],
            device_id=(left,), device_id_type=pl.DeviceIdType.MESH)
        rd_l[0].start()

        for k in range(N_WCH):
            cpw[k].wait()
            w8_ref[pl.ds(k * WCH, WCH), :] = (
                stage_w[k % 2].astype(jnp.float8_e5m2))
            nxt = k + 2
            if nxt < N_WCH:
                cpw[nxt] = pltpu.make_async_copy(
                    w_hbm.at[pl.ds(nxt * WCH, WCH), :], stage_w.at[nxt % 2],
                    w_sems.at[nxt % 2])
                cpw[nxt].start()

        scale_v = s_ref[0]
        slot_busy = [False, False]
        slot_idx = [0]

        def emit(lhs_fp8, row0):
            s = slot_idx[0]
            if slot_busy[s]:
                pltpu.make_async_copy(
                    stage_out.at[s], out_ref.at[pl.ds(0, HALF), :],
                    out_sems.at[s]).wait()
            stage_out[s, :, :] = lax.dot_general(
                lhs_fp8, w8_ref[...],
                (((1,), (0,)), ((), ())),
                preferred_element_type=jnp.float32) * scale_v
            pltpu.make_async_copy(
                stage_out.at[s], out_ref.at[pl.ds(row0, HALF), :],
                out_sems.at[s]).start()
            slot_busy[s] = True
            slot_idx[0] = 1 - s

        emit(x8_ref[pl.ds(0, HALF), :], my * M_PER)
        emit(x8_ref[pl.ds(HALF, HALF), :], my * M_PER + HALF)

        for h in range(N_HOP):
            rd_r[h].wait_recv()
            if h + 1 < N_HOP:
                rd_r[h + 1] = pltpu.make_async_remote_copy(
                    src_ref=comm_r.at[h], dst_ref=comm_r.at[h + 1],
                    send_sem=send_r.at[h + 1], recv_sem=recv_r.at[h + 1],
                    device_id=(right,), device_id_type=pl.DeviceIdType.MESH)
                rd_r[h + 1].start()
            rd_l[h].wait_recv()
            if h + 1 < N_HOP:
                rd_l[h + 1] = pltpu.make_async_remote_copy(
                    src_ref=comm_l.at[h], dst_ref=comm_l.at[h + 1],
                    send_sem=send_l.at[h + 1], recv_sem=recv_l.at[h + 1],
                    device_id=(left,), device_id_type=pl.DeviceIdType.MESH)
                rd_l[h + 1].start()

            origin_r = lax.rem(my + (N_DEV - 1 - h), N_DEV)
            emit(comm_r[h], origin_r * M_PER)
            origin_l = lax.rem(my + 1 + h, N_DEV)
            emit(comm_l[h], origin_l * M_PER + HALF)

        for s in range(2):
            if slot_busy[s]:
                pltpu.make_async_copy(
                    stage_out.at[s], out_ref.at[pl.ds(0, HALF), :],
                    out_sems.at[s]).wait()
        for h in range(N_HOP):
            rd_r[h].wait_send()
            rd_l[h].wait_send()

    out = pl.pallas_call(
        body,
        out_shape=jax.ShapeDtypeStruct((N_DEV * M_PER, N_PER), jnp.float32),
        in_specs=[
            pl.BlockSpec(memory_space=pl.ANY),
            pl.BlockSpec(memory_space=pl.ANY),
            pl.BlockSpec(memory_space=pltpu.SMEM),
        ],
        out_specs=pl.BlockSpec(memory_space=pl.ANY),
        scratch_shapes=[
            pltpu.VMEM((M_PER, K), jnp.float32),
            pltpu.VMEM((M_PER, K), jnp.float8_e4m3fn),
            pltpu.VMEM((2, WCH, N_PER), jnp.float32),
            pltpu.VMEM((K, N_PER), jnp.float8_e5m2),
            pltpu.VMEM((N_HOP, HALF, K), jnp.float8_e4m3fn),
            pltpu.VMEM((N_HOP, HALF, K), jnp.float8_e4m3fn),
            pltpu.VMEM((2, HALF, N_PER), jnp.float32),
            pltpu.SemaphoreType.DMA((2,)),
            pltpu.SemaphoreType.DMA((2,)),
            pltpu.SemaphoreType.DMA((N_HOP,)),
            pltpu.SemaphoreType.DMA((N_HOP,)),
            pltpu.SemaphoreType.DMA((N_HOP,)),
            pltpu.SemaphoreType.DMA((N_HOP,)),
            pltpu.SemaphoreType.DMA((2,)),
        ],
        compiler_params=pltpu.CompilerParams(
            collective_id=0, vmem_limit_bytes=62 * 1024 * 1024),
    )(x, w_mat, scale)
    return out


# device time: 116690 ns/iter; 1.0007x vs baseline; 1.0007x over previous
import jax
import jax.numpy as jnp
from jax import lax
from jax.experimental import pallas as pl
from jax.experimental.pallas import tpu as pltpu

N_DEV = 4
M_PER = 1024
HALF = 512
K = 4096
N_PER = 2048
N_HOP = N_DEV - 1
WCH = 512
N_WCH = K // WCH


def kernel(x, w_mat, scale_x, scale_w):
    scale = (scale_x * scale_w).astype(jnp.float32)

    def body(x_hbm, w_hbm, s_ref, out_ref,
             stage_x, x8_ref, stage_w, w8_ref, comm_r, comm_l, stage_out,
             in_sems, w_sems, send_r, recv_r, send_l, recv_l, out_sems):
        my = lax.axis_index("i")
        right = lax.rem(my + 1, N_DEV)
        left = lax.rem(my + N_DEV - 1, N_DEV)

        barrier_sem = pltpu.get_barrier_semaphore()
        pl.semaphore_signal(barrier_sem, inc=1, device_id=(left,),
                            device_id_type=pl.DeviceIdType.MESH)
        pl.semaphore_signal(barrier_sem, inc=1, device_id=(right,),
                            device_id_type=pl.DeviceIdType.MESH)

        cpx = [pltpu.make_async_copy(
                   x_hbm.at[pl.ds(i * HALF, HALF), :],
                   stage_x.at[pl.ds(i * HALF, HALF), :],
                   in_sems.at[i]) for i in range(2)]
        cpx[0].start()
        cpx[1].start()

        cpw = [None] * N_WCH
        for k in range(2):
            cpw[k] = pltpu.make_async_copy(
                w_hbm.at[pl.ds(k * WCH, WCH), :], stage_w.at[k % 2],
                w_sems.at[k % 2])
            cpw[k].start()

        pl.semaphore_wait(barrier_sem, 2)

        rd_r = [None] * N_HOP
        rd_l = [None] * N_HOP
        cpx[0].wait()
        x8_ref[pl.ds(0, HALF), :] = (
            stage_x[pl.ds(0, HALF), :].astype(jnp.float8_e4m3fn))
        rd_r[0] = pltpu.make_async_remote_copy(
            src_ref=x8_ref.at[pl.ds(0, HALF)],
            dst_ref=comm_r.at[0],
            send_sem=send_r.at[0], recv_sem=recv_r.at[0],
            device_id=(right,), device_id_type=pl.DeviceIdType.MESH)
        rd_r[0].start()
        cpx[1].wait()
        x8_ref[pl.ds(HALF, HALF), :] = (
            stage_x[pl.ds(HALF, HALF), :].astype(jnp.float8_e4m3fn))
        rd_l[0] = pltpu.make_async_remote_copy(
            src_ref=x8_ref.at[pl.ds(HALF, HALF)],
            dst_ref=comm_l.at[0],
            send_sem=send_l.at[0], recv_sem=recv_l.at[0],
            device_id=(left,), device_id_type=pl.DeviceIdType.MESH)
        rd_l[0].start()

        def w_chunk(k):
            cpw[k].wait()
            w8_ref[pl.ds(k * WCH, WCH), :] = (
                stage_w[k % 2].astype(jnp.float8_e5m2))
            nxt = k + 2
            if nxt < N_WCH:
                cpw[nxt] = pltpu.make_async_copy(
                    w_hbm.at[pl.ds(nxt * WCH, WCH), :], stage_w.at[nxt % 2],
                    w_sems.at[nxt % 2])
                cpw[nxt].start()

        scale_v = s_ref[0]
        slot_busy = [False, False]
        slot_idx = [0]

        def emit(lhs_fp8, row0):
            s = slot_idx[0]
            if slot_busy[s]:
                pltpu.make_async_copy(
                    stage_out.at[s], out_ref.at[pl.ds(0, HALF), :],
                    out_sems.at[s]).wait()
            stage_out[s, :, :] = lax.dot_general(
                lhs_fp8, w8_ref[...],
                (((1,), (0,)), ((), ())),
                preferred_element_type=jnp.float32) * scale_v
            pltpu.make_async_copy(
                stage_out.at[s], out_ref.at[pl.ds(row0, HALF), :],
                out_sems.at[s]).start()
            slot_busy[s] = True
            slot_idx[0] = 1 - s

        def hop_wait_and_forward(h):
            rd_r[h].wait_recv()
            if h + 1 < N_HOP:
                rd_r[h + 1] = pltpu.make_async_remote_copy(
                    src_ref=comm_r.at[h], dst_ref=comm_r.at[h + 1],
                    send_sem=send_r.at[h + 1], recv_sem=recv_r.at[h + 1],
                    device_id=(right,), device_id_type=pl.DeviceIdType.MESH)
                rd_r[h + 1].start()
            rd_l[h].wait_recv()
            if h + 1 < N_HOP:
                rd_l[h + 1] = pltpu.make_async_remote_copy(
                    src_ref=comm_l.at[h], dst_ref=comm_l.at[h + 1],
                    send_sem=send_l.at[h + 1], recv_sem=recv_l.at[h + 1],
                    device_id=(left,), device_id_type=pl.DeviceIdType.MESH)
                rd_l[h + 1].start()

        def hop_emit(h):
            origin_r = lax.rem(my + (N_DEV - 1 - h), N_DEV)
            emit(comm_r[h], origin_r * M_PER)
            origin_l = lax.rem(my + 1 + h, N_DEV)
            emit(comm_l[h], origin_l * M_PER + HALF)

        for k in range(5):
            w_chunk(k)
        hop_wait_and_forward(0)
        for k in range(5, N_WCH):
            w_chunk(k)
        emit(x8_ref[pl.ds(0, HALF), :], my * M_PER)
        emit(x8_ref[pl.ds(HALF, HALF), :], my * M_PER + HALF)
        hop_emit(0)
        hop_wait_and_forward(1)
        hop_emit(1)
        hop_wait_and_forward(2)
        hop_emit(2)

        for s in range(2):
            if slot_busy[s]:
                pltpu.make_async_copy(
                    stage_out.at[s], out_ref.at[pl.ds(0, HALF), :],
                    out_sems.at[s]).wait()
        for h in range(N_HOP):
            rd_r[h].wait_send()
            rd_l[h].wait_send()

    out = pl.pallas_call(
        body,
        out_shape=jax.ShapeDtypeStruct((N_DEV * M_PER, N_PER), jnp.float32),
        in_specs=[
            pl.BlockSpec(memory_space=pl.ANY),
            pl.BlockSpec(memory_space=pl.ANY),
            pl.BlockSpec(memory_space=pltpu.SMEM),
        ],
        out_specs=pl.BlockSpec(memory_space=pl.ANY),
        scratch_shapes=[
            pltpu.VMEM((M_PER, K), jnp.float32),
            pltpu.VMEM((M_PER, K), jnp.float8_e4m3fn),
            pltpu.VMEM((2, WCH, N_PER), jnp.float32),
            pltpu.VMEM((K, N_PER), jnp.float8_e5m2),
            pltpu.VMEM((N_HOP, HALF, K), jnp.float8_e4m3fn),
            pltpu.VMEM((N_HOP, HALF, K), jnp.float8_e4m3fn),
            pltpu.VMEM((2, HALF, N_PER), jnp.float32),
            pltpu.SemaphoreType.DMA((2,)),
            pltpu.SemaphoreType.DMA((2,)),
            pltpu.SemaphoreType.DMA((N_HOP,)),
            pltpu.SemaphoreType.DMA((N_HOP,)),
            pltpu.SemaphoreType.DMA((N_HOP,)),
            pltpu.SemaphoreType.DMA((N_HOP,)),
            pltpu.SemaphoreType.DMA((2,)),
        ],
        compiler_params=pltpu.CompilerParams(
            collective_id=0, vmem_limit_bytes=62 * 1024 * 1024),
    )(x, w_mat, scale)
    return out


# device time: 116593 ns/iter; 1.0015x vs baseline; 1.0008x over previous
import jax
import jax.numpy as jnp
from jax import lax
from jax.experimental import pallas as pl
from jax.experimental.pallas import tpu as pltpu

N_DEV = 4
M_PER = 1024
HALF = 512
QTR = 256
K = 4096
N_PER = 2048
N_HOP = N_DEV - 1
WCH = 512
N_WCH = K // WCH


def kernel(x, w_mat, scale_x, scale_w):
    scale = (scale_x * scale_w).astype(jnp.float32)

    def body(x_hbm, w_hbm, s_ref, out_ref,
             stage_x, x8_ref, stage_w, w8_ref, comm_r, comm_l,
             stage_big, stage_sml,
             x_sems, w_sems, send_r, recv_r, send_l, recv_l,
             big_sems, sml_sems):
        my = lax.axis_index("i")
        right = lax.rem(my + 1, N_DEV)
        left = lax.rem(my + N_DEV - 1, N_DEV)

        barrier_sem = pltpu.get_barrier_semaphore()
        pl.semaphore_signal(barrier_sem, inc=1, device_id=(left,),
                            device_id_type=pl.DeviceIdType.MESH)
        pl.semaphore_signal(barrier_sem, inc=1, device_id=(right,),
                            device_id_type=pl.DeviceIdType.MESH)

        cpx = [pltpu.make_async_copy(
                   x_hbm.at[pl.ds(q * QTR, QTR), :],
                   stage_x.at[pl.ds(q * QTR, QTR), :],
                   x_sems.at[q]) for q in range(4)]
        for q in range(4):
            cpx[q].start()
        cpw = [None] * N_WCH
        for k in range(2):
            cpw[k] = pltpu.make_async_copy(
                w_hbm.at[pl.ds(k * WCH, WCH), :], stage_w.at[k % 2],
                w_sems.at[k % 2])
            cpw[k].start()

        pl.semaphore_wait(barrier_sem, 2)

        rd_r = [[None] * 2 for _ in range(N_HOP)]
        rd_l = [[None] * 2 for _ in range(N_HOP)]
        for q in range(4):
            cpx[q].wait()
            x8_ref[pl.ds(q * QTR, QTR), :] = (
                stage_x[pl.ds(q * QTR, QTR), :].astype(jnp.float8_e4m3fn))
            s = q % 2
            if q < 2:
                rd_r[0][s] = pltpu.make_async_remote_copy(
                    src_ref=x8_ref.at[pl.ds(q * QTR, QTR)],
                    dst_ref=comm_r.at[0, pl.ds(s * QTR, QTR)],
                    send_sem=send_r.at[0, s], recv_sem=recv_r.at[0, s],
                    device_id=(right,), device_id_type=pl.DeviceIdType.MESH)
                rd_r[0][s].start()
            else:
                rd_l[0][s] = pltpu.make_async_remote_copy(
                    src_ref=x8_ref.at[pl.ds(q * QTR, QTR)],
                    dst_ref=comm_l.at[0, pl.ds(s * QTR, QTR)],
                    send_sem=send_l.at[0, s], recv_sem=recv_l.at[0, s],
                    device_id=(left,), device_id_type=pl.DeviceIdType.MESH)
                rd_l[0][s].start()

        def w_chunk(k):
            cpw[k].wait()
            w8_ref[pl.ds(k * WCH, WCH), :] = (
                stage_w[k % 2].astype(jnp.float8_e5m2))
            nxt = k + 2
            if nxt < N_WCH:
                cpw[nxt] = pltpu.make_async_copy(
                    w_hbm.at[pl.ds(nxt * WCH, WCH), :], stage_w.at[nxt % 2],
                    w_sems.at[nxt % 2])
                cpw[nxt].start()

        def fwd(h, s):
            rr = pltpu.make_async_remote_copy(
                src_ref=comm_r.at[h - 1, pl.ds(s * QTR, QTR)],
                dst_ref=comm_r.at[h, pl.ds(s * QTR, QTR)],
                send_sem=send_r.at[h, s], recv_sem=recv_r.at[h, s],
                device_id=(right,), device_id_type=pl.DeviceIdType.MESH)
            ll = pltpu.make_async_remote_copy(
                src_ref=comm_l.at[h - 1, pl.ds(s * QTR, QTR)],
                dst_ref=comm_l.at[h, pl.ds(s * QTR, QTR)],
                send_sem=send_l.at[h, s], recv_sem=recv_l.at[h, s],
                device_id=(left,), device_id_type=pl.DeviceIdType.MESH)
            rd_r[h][s] = rr
            rd_l[h][s] = ll

        def hop_waits_and_forwards(h):
            for s in range(2):
                rd_r[h][s].wait_recv()
                if h + 1 < N_HOP:
                    fwd(h + 1, s)
                    rd_r[h + 1][s].start()
                rd_l[h][s].wait_recv()
                if h + 1 < N_HOP:
                    rd_l[h + 1][s].start()

        scale_v = s_ref[0]
        big_state = [False, False, 0]
        sml_state = [False, False, 0]

        def _emit(stage, sems, state, rows, lhs_fp8, row0):
            s = state[2]
            if state[s]:
                pltpu.make_async_copy(
                    stage.at[s], out_ref.at[pl.ds(0, rows), :],
                    sems.at[s]).wait()
            stage[s, :, :] = lax.dot_general(
                lhs_fp8, w8_ref[...],
                (((1,), (0,)), ((), ())),
                preferred_element_type=jnp.float32) * scale_v
            pltpu.make_async_copy(
                stage.at[s], out_ref.at[pl.ds(row0, rows), :],
                sems.at[s]).start()
            state[s] = True
            state[2] = 1 - s

        def emit_big(lhs_fp8, row0):
            _emit(stage_big, big_sems, big_state, HALF, lhs_fp8, row0)

        def emit_sml(lhs_fp8, row0):
            _emit(stage_sml, sml_sems, sml_state, QTR, lhs_fp8, row0)

        for k in range(5):
            w_chunk(k)
        hop_waits_and_forwards(0)
        for k in range(5, N_WCH):
            w_chunk(k)
        hop_waits_and_forwards(1)
        emit_big(x8_ref[pl.ds(0, HALF), :], my * M_PER)
        emit_big(x8_ref[pl.ds(HALF, HALF), :], my * M_PER + HALF)
        for h in range(2):
            origin_r = lax.rem(my + (N_DEV - 1 - h), N_DEV)
            emit_big(comm_r[h], origin_r * M_PER)
            origin_l = lax.rem(my + 1 + h, N_DEV)
            emit_big(comm_l[h], origin_l * M_PER + HALF)
        origin_r2 = lax.rem(my + 1, N_DEV)
        origin_l2 = lax.rem(my + 3, N_DEV)
        for s in range(2):
            rd_r[2][s].wait_recv()
            emit_sml(comm_r[2, pl.ds(s * QTR, QTR), :],
                     origin_r2 * M_PER + s * QTR)
            rd_l[2][s].wait_recv()
            emit_sml(comm_l[2, pl.ds(s * QTR, QTR), :],
                     origin_l2 * M_PER + HALF + s * QTR)

        for st, sems, state, rows in ((stage_big, big_sems, big_state, HALF),
                                      (stage_sml, sml_sems, sml_state, QTR)):
            for s in range(2):
                if state[s]:
                    pltpu.make_async_copy(
                        st.at[s], out_ref.at[pl.ds(0, rows), :],
                        sems.at[s]).wait()
        for h in range(N_HOP):
            for s in range(2):
                rd_r[h][s].wait_send()
                rd_l[h][s].wait_send()

    out = pl.pallas_call(
        body,
        out_shape=jax.ShapeDtypeStruct((N_DEV * M_PER, N_PER), jnp.float32),
        in_specs=[
            pl.BlockSpec(memory_space=pl.ANY),
            pl.BlockSpec(memory_space=pl.ANY),
            pl.BlockSpec(memory_space=pltpu.SMEM),
        ],
        out_specs=pl.BlockSpec(memory_space=pl.ANY),
        scratch_shapes=[
            pltpu.VMEM((M_PER, K), jnp.float32),
            pltpu.VMEM((M_PER, K), jnp.float8_e4m3fn),
            pltpu.VMEM((2, WCH, N_PER), jnp.float32),
            pltpu.VMEM((K, N_PER), jnp.float8_e5m2),
            pltpu.VMEM((N_HOP, HALF, K), jnp.float8_e4m3fn),
            pltpu.VMEM((N_HOP, HALF, K), jnp.float8_e4m3fn),
            pltpu.VMEM((2, HALF, N_PER), jnp.float32),
            pltpu.VMEM((2, QTR, N_PER), jnp.float32),
            pltpu.SemaphoreType.DMA((4,)),
            pltpu.SemaphoreType.DMA((2,)),
            pltpu.SemaphoreType.DMA((N_HOP, 2)),
            pltpu.SemaphoreType.DMA((N_HOP, 2)),
            pltpu.SemaphoreType.DMA((N_HOP, 2)),
            pltpu.SemaphoreType.DMA((N_HOP, 2)),
            pltpu.SemaphoreType.DMA((2,)),
            pltpu.SemaphoreType.DMA((2,)),
        ],
        compiler_params=pltpu.CompilerParams(
            collective_id=0, vmem_limit_bytes=63 * 1024 * 1024),
    )(x, w_mat, scale)
    return out
